# Initial kernel scaffold; baseline (speedup 1.0000x reference)
import jax
import jax.numpy as jnp
from jax.experimental import pallas as pl


def kernel(x, gamma, beta):
    raise NotImplementedError("write your pallas kernel here")


# speedup vs baseline: 1.2904x; 1.2904x over previous
import functools

import jax
import jax.numpy as jnp
from jax import lax
from jax.experimental import pallas as pl
from jax.experimental.pallas import tpu as pltpu
from jax.experimental.pallas import tpu_sc as plsc

B, T, D = 4, 8192, 1024
NUM_PATTERN = 8
PATTERN_LEN = T // 4 // NUM_PATTERN
HALF = PATTERN_LEN // 2
N_COMP = NUM_PATTERN * PATTERN_LEN * D
EPS = 1e-8

L = 16
NSHARD = 8
SH = T // NSHARD
NCHUNK = SH // L


def _row_stats(x):
    TTA = 1024

    def body(x_ref, rs_ref, rsq_ref):
        xb = x_ref[0]
        rs_ref[0] = jnp.sum(xb, axis=1, keepdims=True)
        rsq_ref[0] = jnp.sum(xb * xb, axis=1, keepdims=True)

    return pl.pallas_call(
        body,
        grid=(B, T // TTA),
        in_specs=[pl.BlockSpec((1, TTA, D), lambda b, i: (b, i, 0))],
        out_specs=[
            pl.BlockSpec((1, TTA, 1), lambda b, i: (b, i, 0)),
            pl.BlockSpec((1, TTA, 1), lambda b, i: (b, i, 0)),
        ],
        out_shape=[
            jax.ShapeDtypeStruct((B, T, 1), jnp.float32),
            jax.ShapeDtypeStruct((B, T, 1), jnp.float32),
        ],
        compiler_params=pltpu.CompilerParams(
            dimension_semantics=("parallel", "parallel")),
    )(x)


def _sc_stage(rs_flat, rsq_flat):
    mesh = plsc.VectorSubcoreMesh(core_axis_name="c", subcore_axis_name="s")

    @functools.partial(
        pl.kernel,
        mesh=mesh,
        out_type=[
            jax.ShapeDtypeStruct((B, L), jnp.float32),
            jax.ShapeDtypeStruct((B, L), jnp.int32),
        ],
        scratch_types=[
            pltpu.VMEM((SH + 2 * L,), jnp.float32),
            pltpu.VMEM((SH,), jnp.float32),
            pltpu.VMEM((T,), jnp.float32),
            pltpu.VMEM((T,), jnp.float32),
            pltpu.VMEM((L,), jnp.float32),
            pltpu.VMEM((L,), jnp.int32),
            pltpu.VMEM((NSHARD * L,), jnp.float32),
            pltpu.VMEM((NSHARD * L,), jnp.int32),
            pltpu.VMEM_SHARED((16 * L,), jnp.float32),
            pltpu.VMEM_SHARED((16 * L,), jnp.int32),
        ],
        compiler_params=pltpu.CompilerParams(needs_layout_passes=False),
    )
    def sc_kernel(rs_hbm, rsq_hbm, stats_hbm, peaks_hbm,
                  halo_v, xp_v, rs_full, rsq_full, stg_f, stg_i,
                  mv, mi, shv, shi):
        c = lax.axis_index("c")
        s = lax.axis_index("s")
        b = c * 2 + s // NSHARD
        shard = s % NSHARD
        t0 = shard * SH
        base = b * T + t0
        lanes = lax.iota(jnp.int32, L)
        neg_inf = jnp.float32(-jnp.inf)
        big_i = jnp.int32(2**30)

        halo_v[pl.ds(0, L)] = jnp.full((L,), neg_inf, jnp.float32)
        halo_v[pl.ds(SH + L, L)] = jnp.full((L,), neg_inf, jnp.float32)
        pltpu.sync_copy(rs_hbm.at[pl.ds(base, SH)], halo_v.at[pl.ds(L, SH)])

        @pl.when(shard > 0)
        def _():
            pltpu.sync_copy(rs_hbm.at[pl.ds(base - L, L)],
                            halo_v.at[pl.ds(0, L)])

        @pl.when(shard < NSHARD - 1)
        def _():
            pltpu.sync_copy(rs_hbm.at[pl.ds(base + SH, L)],
                            halo_v.at[pl.ds(SH + L, L)])

        def peak_body(cb, carry):
            pos = cb * L + lanes
            ctr = plsc.load_gather(halo_v, [pos + L])
            lft = plsc.load_gather(halo_v, [pos + (L - 1)])
            rgt = plsc.load_gather(halo_v, [pos + (L + 1)])
            xp = jnp.where((ctr >= lft) & (ctr >= rgt), ctr, jnp.float32(0.0))
            plsc.store_scatter(xp_v, [pos], xp)
            return carry

        lax.fori_loop(0, NCHUNK, peak_body, 0)

        topv = jnp.full((L,), neg_inf, jnp.float32)
        topi = jnp.zeros((L,), jnp.int32)
        for k in range(NUM_PATTERN):
            def amax_body(cb, carry):
                m, mi_ = carry
                pos = cb * L + lanes
                v = plsc.load_gather(xp_v, [pos])
                gi = t0 + pos
                upd = (v > m) | ((v == m) & (gi < mi_))
                return jnp.where(upd, v, m), jnp.where(upd, gi, mi_)

            m, mi_ = lax.fori_loop(
                0, NCHUNK, amax_body,
                (jnp.full((L,), neg_inf, jnp.float32),
                 jnp.full((L,), big_i, jnp.int32)))
            maxv = jnp.max(m)
            gidx = jnp.min(jnp.where(m == maxv, mi_, big_i))
            topv = jnp.where(lanes == k, maxv, topv)
            topi = jnp.where(lanes == k, gidx, topi)
            plsc.store_scatter(xp_v, [jnp.zeros((L,), jnp.int32) + (gidx - t0)],
                               jnp.full((L,), neg_inf, jnp.float32),
                               mask=lanes == 0)

        stg_f[...] = topv
        stg_i[...] = topi
        pltpu.sync_copy(stg_f, shv.at[pl.ds(s * L, L)])
        pltpu.sync_copy(stg_i, shi.at[pl.ds(s * L, L)])
        plsc.subcore_barrier()

        @pl.when(shard == 0)
        def _():
            pltpu.sync_copy(shv.at[pl.ds((s // NSHARD) * NSHARD * L, NSHARD * L)], mv)
            pltpu.sync_copy(shi.at[pl.ds((s // NSHARD) * NSHARD * L, NSHARD * L)], mi)

            gtopv = jnp.full((L,), neg_inf, jnp.float32)
            gtopi = jnp.zeros((L,), jnp.int32)
            for k in range(NUM_PATTERN):
                def mrg_body(cb, carry):
                    m, mi_ = carry
                    pos = cb * L + lanes
                    v = plsc.load_gather(mv, [pos])
                    gi = plsc.load_gather(mi, [pos])
                    upd = (v > m) | ((v == m) & (gi < mi_))
                    return jnp.where(upd, v, m), jnp.where(upd, gi, mi_)

                m, mi_ = lax.fori_loop(
                    0, NSHARD, mrg_body,
                    (jnp.full((L,), neg_inf, jnp.float32),
                     jnp.full((L,), big_i, jnp.int32)))
                maxv = jnp.max(m)
                gidx = jnp.min(jnp.where(m == maxv, mi_, big_i))
                gtopv = jnp.where(lanes == k, maxv, gtopv)
                gtopi = jnp.where(lanes == k, gidx, gtopi)

                def clr_body(cb, carry):
                    pos = cb * L + lanes
                    v = plsc.load_gather(mv, [pos])
                    gi = plsc.load_gather(mi, [pos])
                    hit = (v == maxv) & (gi == gidx)
                    plsc.store_scatter(mv, [pos],
                                       jnp.full((L,), neg_inf, jnp.float32),
                                       mask=hit)
                    return carry

                lax.fori_loop(0, NSHARD, clr_body, 0)

            pltpu.sync_copy(rs_hbm.at[pl.ds(b * T, T)], rs_full)
            pltpu.sync_copy(rsq_hbm.at[pl.ds(b * T, T)], rsq_full)
            acc1 = jnp.zeros((L,), jnp.float32)
            acc2 = jnp.zeros((L,), jnp.float32)
            for k in range(NUM_PATTERN):
                pk = jnp.max(jnp.where(lanes == k, gtopi,
                                       jnp.int32(-2**31 + 1)))

                def win_body(jc, carry):
                    a1, a2 = carry
                    idxv = jnp.clip(pk - HALF + jc * L + lanes, 0, T - 1)
                    a1 = a1 + plsc.load_gather(rs_full, [idxv])
                    a2 = a2 + plsc.load_gather(rsq_full, [idxv])
                    return a1, a2

                acc1, acc2 = lax.fori_loop(0, PATTERN_LEN // L, win_body,
                                           (acc1, acc2))
            s1 = jnp.sum(acc1)
            s2 = jnp.sum(acc2)

            stg_f[...] = jnp.where(lanes == 0, s1,
                                   jnp.where(lanes == 1, s2,
                                             jnp.float32(0.0)))
            stg_i[...] = gtopi
            pltpu.sync_copy(stg_f, stats_hbm.at[b])
            pltpu.sync_copy(stg_i, peaks_hbm.at[b])

    return sc_kernel(rs_flat, rsq_flat)


def _normalize(x, gamma2, beta2, stats, peaks):
    TTB = 1024

    def body(stats_ref, peaks_ref, x_ref, g_ref, bt_ref, y_ref):
        b = pl.program_id(0)
        i = pl.program_id(1)
        s1 = stats_ref[b, 0]
        s2 = stats_ref[b, 1]
        n = jnp.float32(N_COMP)
        cmean = s1 / n
        var = (s2 - s1 * s1 / n) / jnp.float32(N_COMP - 1)
        inv = 1.0 / (jnp.sqrt(var) + jnp.float32(EPS))
        scale = g_ref[...] * inv
        bias = bt_ref[...]
        tv = i * TTB + lax.broadcasted_iota(jnp.int32, (TTB, 1), 0)
        sel = jnp.zeros((TTB, 1), jnp.bool_)
        for k in range(NUM_PATTERN):
            p = peaks_ref[b, k]
            lo = jnp.maximum(p - HALF, 0)
            hi = jnp.minimum(p + (PATTERN_LEN - 1 - HALF), T - 1)
            sel = sel | ((tv >= lo) & (tv <= hi))
        m = jax.nn.sigmoid(jnp.where(sel, jnp.float32(2.0),
                                     jnp.float32(-3.0)))
        xb = x_ref[0]
        y_ref[0] = ((xb - cmean) * scale + bias) * m

    return pl.pallas_call(
        body,
        grid=(B, T // TTB),
        in_specs=[
            pl.BlockSpec(memory_space=pltpu.SMEM),
            pl.BlockSpec(memory_space=pltpu.SMEM),
            pl.BlockSpec((1, TTB, D), lambda b, i: (b, i, 0)),
            pl.BlockSpec((1, D), lambda b, i: (0, 0)),
            pl.BlockSpec((1, D), lambda b, i: (0, 0)),
        ],
        out_specs=pl.BlockSpec((1, TTB, D), lambda b, i: (b, i, 0)),
        out_shape=jax.ShapeDtypeStruct((B, T, D), jnp.float32),
        compiler_params=pltpu.CompilerParams(
            dimension_semantics=("parallel", "parallel")),
    )(stats, peaks, x, gamma2, beta2)


def kernel(x, gamma, beta):
    rs3, rsq3 = _row_stats(x)
    stats, peaks = _sc_stage(rs3.reshape(B * T), rsq3.reshape(B * T))
    return _normalize(x, gamma.reshape(1, D), beta.reshape(1, D),
                      stats, peaks)


# speedup vs baseline: 1.3133x; 1.0177x over previous
import functools

import jax
import jax.numpy as jnp
from jax import lax
from jax.experimental import pallas as pl
from jax.experimental.pallas import tpu as pltpu
from jax.experimental.pallas import tpu_sc as plsc

B, T, D = 4, 8192, 1024
NUM_PATTERN = 8
PATTERN_LEN = T // 4 // NUM_PATTERN
HALF = PATTERN_LEN // 2
N_COMP = NUM_PATTERN * PATTERN_LEN * D
EPS = 1e-8

L = 16
NSHARD = 8
SH = T // NSHARD
NCHUNK = SH // L


def _row_stats(x):
    TTA = 2048

    def body(x_ref, rs_ref, rsq_ref):
        xb = x_ref[0]
        rs_ref[0] = jnp.sum(xb, axis=1, keepdims=True)
        rsq_ref[0] = jnp.sum(xb * xb, axis=1, keepdims=True)

    return pl.pallas_call(
        body,
        grid=(B, T // TTA),
        in_specs=[pl.BlockSpec((1, TTA, D), lambda b, i: (b, i, 0))],
        out_specs=[
            pl.BlockSpec((1, TTA, 1), lambda b, i: (b, i, 0)),
            pl.BlockSpec((1, TTA, 1), lambda b, i: (b, i, 0)),
        ],
        out_shape=[
            jax.ShapeDtypeStruct((B, T, 1), jnp.float32),
            jax.ShapeDtypeStruct((B, T, 1), jnp.float32),
        ],
        compiler_params=pltpu.CompilerParams(
            dimension_semantics=("parallel", "parallel")),
    )(x)


def _sc_stage(rs_flat, rsq_flat):
    mesh = plsc.VectorSubcoreMesh(core_axis_name="c", subcore_axis_name="s")

    @functools.partial(
        pl.kernel,
        mesh=mesh,
        out_type=[
            jax.ShapeDtypeStruct((B, L), jnp.float32),
            jax.ShapeDtypeStruct((B * T,), jnp.float32),
        ],
        scratch_types=[
            pltpu.VMEM((SH + 2 * L,), jnp.float32),
            pltpu.VMEM((SH,), jnp.float32),
            pltpu.VMEM((T,), jnp.float32),
            pltpu.VMEM((T,), jnp.float32),
            pltpu.VMEM((L,), jnp.float32),
            pltpu.VMEM((L,), jnp.int32),
            pltpu.VMEM((NSHARD * L,), jnp.float32),
            pltpu.VMEM((NSHARD * L,), jnp.int32),
            pltpu.VMEM_SHARED((16 * L,), jnp.float32),
            pltpu.VMEM_SHARED((16 * L,), jnp.int32),
            pltpu.VMEM_SHARED((2 * L,), jnp.int32),
        ],
        compiler_params=pltpu.CompilerParams(needs_layout_passes=False),
    )
    def sc_kernel(rs_hbm, rsq_hbm, stats_hbm, mask_hbm,
                  halo_v, xp_v, rs_full, rsq_full, stg_f, stg_i,
                  mv, mi, shv, shi, shp):
        c = lax.axis_index("c")
        s = lax.axis_index("s")
        b = c * 2 + s // NSHARD
        shard = s % NSHARD
        t0 = shard * SH
        base = b * T + t0
        lanes = lax.iota(jnp.int32, L)
        neg_inf = jnp.float32(-jnp.inf)
        big_i = jnp.int32(2**30)

        halo_v[pl.ds(0, L)] = jnp.full((L,), neg_inf, jnp.float32)
        halo_v[pl.ds(SH + L, L)] = jnp.full((L,), neg_inf, jnp.float32)
        pltpu.sync_copy(rs_hbm.at[pl.ds(base, SH)], halo_v.at[pl.ds(L, SH)])

        @pl.when(shard > 0)
        def _():
            pltpu.sync_copy(rs_hbm.at[pl.ds(base - L, L)],
                            halo_v.at[pl.ds(0, L)])

        @pl.when(shard < NSHARD - 1)
        def _():
            pltpu.sync_copy(rs_hbm.at[pl.ds(base + SH, L)],
                            halo_v.at[pl.ds(SH + L, L)])

        def peak_body(cb, carry):
            pos = cb * L + lanes
            ctr = plsc.load_gather(halo_v, [pos + L])
            lft = plsc.load_gather(halo_v, [pos + (L - 1)])
            rgt = plsc.load_gather(halo_v, [pos + (L + 1)])
            xp = jnp.where((ctr >= lft) & (ctr >= rgt), ctr, jnp.float32(0.0))
            plsc.store_scatter(xp_v, [pos], xp)
            return carry

        lax.fori_loop(0, NCHUNK, peak_body, 0)

        topv = jnp.full((L,), neg_inf, jnp.float32)
        topi = jnp.zeros((L,), jnp.int32)
        for k in range(NUM_PATTERN):
            def amax_body(cb, carry):
                m, mi_ = carry
                pos = cb * L + lanes
                v = plsc.load_gather(xp_v, [pos])
                gi = t0 + pos
                upd = (v > m) | ((v == m) & (gi < mi_))
                return jnp.where(upd, v, m), jnp.where(upd, gi, mi_)

            m, mi_ = lax.fori_loop(
                0, NCHUNK, amax_body,
                (jnp.full((L,), neg_inf, jnp.float32),
                 jnp.full((L,), big_i, jnp.int32)))
            maxv = jnp.max(m)
            gidx = jnp.min(jnp.where(m == maxv, mi_, big_i))
            topv = jnp.where(lanes == k, maxv, topv)
            topi = jnp.where(lanes == k, gidx, topi)
            plsc.store_scatter(xp_v, [jnp.zeros((L,), jnp.int32) + (gidx - t0)],
                               jnp.full((L,), neg_inf, jnp.float32),
                               mask=lanes == 0)

        stg_f[...] = topv
        stg_i[...] = topi
        pltpu.sync_copy(stg_f, shv.at[pl.ds(s * L, L)])
        pltpu.sync_copy(stg_i, shi.at[pl.ds(s * L, L)])
        plsc.subcore_barrier()

        @pl.when(shard == 0)
        def _():
            pltpu.sync_copy(shv.at[pl.ds((s // NSHARD) * NSHARD * L, NSHARD * L)], mv)
            pltpu.sync_copy(shi.at[pl.ds((s // NSHARD) * NSHARD * L, NSHARD * L)], mi)

            gtopv = jnp.full((L,), neg_inf, jnp.float32)
            gtopi = jnp.zeros((L,), jnp.int32)
            for k in range(NUM_PATTERN):
                def mrg_body(cb, carry):
                    m, mi_ = carry
                    pos = cb * L + lanes
                    v = plsc.load_gather(mv, [pos])
                    gi = plsc.load_gather(mi, [pos])
                    upd = (v > m) | ((v == m) & (gi < mi_))
                    return jnp.where(upd, v, m), jnp.where(upd, gi, mi_)

                m, mi_ = lax.fori_loop(
                    0, NSHARD, mrg_body,
                    (jnp.full((L,), neg_inf, jnp.float32),
                     jnp.full((L,), big_i, jnp.int32)))
                maxv = jnp.max(m)
                gidx = jnp.min(jnp.where(m == maxv, mi_, big_i))
                gtopv = jnp.where(lanes == k, maxv, gtopv)
                gtopi = jnp.where(lanes == k, gidx, gtopi)

                def clr_body(cb, carry):
                    pos = cb * L + lanes
                    v = plsc.load_gather(mv, [pos])
                    gi = plsc.load_gather(mi, [pos])
                    hit = (v == maxv) & (gi == gidx)
                    plsc.store_scatter(mv, [pos],
                                       jnp.full((L,), neg_inf, jnp.float32),
                                       mask=hit)
                    return carry

                lax.fori_loop(0, NSHARD, clr_body, 0)

            pltpu.sync_copy(rs_hbm.at[pl.ds(b * T, T)], rs_full)
            pltpu.sync_copy(rsq_hbm.at[pl.ds(b * T, T)], rsq_full)
            acc1 = jnp.zeros((L,), jnp.float32)
            acc2 = jnp.zeros((L,), jnp.float32)
            for k in range(NUM_PATTERN):
                pk = jnp.max(jnp.where(lanes == k, gtopi,
                                       jnp.int32(-2**31 + 1)))

                def win_body(jc, carry):
                    a1, a2 = carry
                    idxv = jnp.clip(pk - HALF + jc * L + lanes, 0, T - 1)
                    a1 = a1 + plsc.load_gather(rs_full, [idxv])
                    a2 = a2 + plsc.load_gather(rsq_full, [idxv])
                    return a1, a2

                acc1, acc2 = lax.fori_loop(0, PATTERN_LEN // L, win_body,
                                           (acc1, acc2))
            s1 = jnp.sum(acc1)
            s2 = jnp.sum(acc2)

            stg_f[...] = jnp.where(lanes == 0, s1,
                                   jnp.where(lanes == 1, s2,
                                             jnp.float32(0.0)))
            stg_i[...] = gtopi
            pltpu.sync_copy(stg_f, stats_hbm.at[b])
            pltpu.sync_copy(stg_i, shp.at[pl.ds((s // NSHARD) * L, L)])

        plsc.subcore_barrier()
        pltpu.sync_copy(shp.at[pl.ds((s // NSHARD) * L, L)], stg_i)
        pks = stg_i[...]
        los = []
        his = []
        for k in range(NUM_PATTERN):
            pk = jnp.max(jnp.where(lanes == k, pks, jnp.int32(-2**31 + 1)))
            los.append(jnp.maximum(pk - HALF, 0))
            his.append(jnp.minimum(pk + (PATTERN_LEN - 1 - HALF), T - 1))

        def mask_body(cb, carry):
            pos = cb * L + lanes
            t = t0 + pos
            inb = (t >= los[0]) & (t <= his[0])
            for k in range(1, NUM_PATTERN):
                inb = inb | ((t >= los[k]) & (t <= his[k]))
            plsc.store_scatter(
                xp_v, [pos],
                jnp.where(inb, jnp.float32(1.0), jnp.float32(0.0)))
            return carry

        lax.fori_loop(0, NCHUNK, mask_body, 0)
        pltpu.sync_copy(xp_v, mask_hbm.at[pl.ds(base, SH)])

    return sc_kernel(rs_flat, rsq_flat)


def _normalize(x, gamma2, beta2, stats, mask3):
    TTB = 2048

    def body(stats_ref, x_ref, m_ref, g_ref, bt_ref, y_ref):
        b = pl.program_id(0)
        s1 = stats_ref[b, 0]
        s2 = stats_ref[b, 1]
        n = jnp.float32(N_COMP)
        cmean = s1 / n
        var = (s2 - s1 * s1 / n) / jnp.float32(N_COMP - 1)
        inv = 1.0 / (jnp.sqrt(var) + jnp.float32(EPS))
        scale = g_ref[...] * inv
        bias = bt_ref[...] - cmean * scale
        c_in = jax.nn.sigmoid(jnp.float32(2.0))
        c_out = jax.nn.sigmoid(jnp.float32(-3.0))
        m = c_out + m_ref[0] * (c_in - c_out)
        xb = x_ref[0]
        y_ref[0] = (xb * scale + bias) * m

    return pl.pallas_call(
        body,
        grid=(B, T // TTB),
        in_specs=[
            pl.BlockSpec(memory_space=pltpu.SMEM),
            pl.BlockSpec((1, TTB, D), lambda b, i: (b, i, 0)),
            pl.BlockSpec((1, TTB, 1), lambda b, i: (b, i, 0)),
            pl.BlockSpec((1, D), lambda b, i: (0, 0)),
            pl.BlockSpec((1, D), lambda b, i: (0, 0)),
        ],
        out_specs=pl.BlockSpec((1, TTB, D), lambda b, i: (b, i, 0)),
        out_shape=jax.ShapeDtypeStruct((B, T, D), jnp.float32),
        compiler_params=pltpu.CompilerParams(
            dimension_semantics=("parallel", "parallel")),
    )(stats, x, mask3, gamma2, beta2)


def kernel(x, gamma, beta):
    rs3, rsq3 = _row_stats(x)
    stats, mask01 = _sc_stage(rs3.reshape(B * T), rsq3.reshape(B * T))
    return _normalize(x, gamma.reshape(1, D), beta.reshape(1, D),
                      stats, mask01.reshape(B, T, 1))


# speedup vs baseline: 1.6373x; 1.2467x over previous
import functools

import jax
import jax.numpy as jnp
from jax import lax
from jax.experimental import pallas as pl
from jax.experimental.pallas import tpu as pltpu
from jax.experimental.pallas import tpu_sc as plsc

B, T, D = 4, 8192, 1024
NUM_PATTERN = 8
PATTERN_LEN = T // 4 // NUM_PATTERN
HALF = PATTERN_LEN // 2
N_COMP = NUM_PATTERN * PATTERN_LEN * D
EPS = 1e-8

L = 16
NSHARD = 8
SH = T // NSHARD
NCHUNK = SH // L


def _row_stats(x):
    TTA = 2048

    def body(x_ref, rs_ref, rsq_ref):
        xb = x_ref[0]
        rs_ref[...] = jnp.sum(xb, axis=1)
        rsq_ref[...] = jnp.sum(xb * xb, axis=1)

    return pl.pallas_call(
        body,
        grid=(B, T // TTA),
        in_specs=[pl.BlockSpec((1, TTA, D), lambda b, i: (b, i, 0))],
        out_specs=[
            pl.BlockSpec((TTA,), lambda b, i: (b * (T // TTA) + i,)),
            pl.BlockSpec((TTA,), lambda b, i: (b * (T // TTA) + i,)),
        ],
        out_shape=[
            jax.ShapeDtypeStruct((B * T,), jnp.float32),
            jax.ShapeDtypeStruct((B * T,), jnp.float32),
        ],
        compiler_params=pltpu.CompilerParams(
            dimension_semantics=("parallel", "parallel")),
    )(x)


def _sc_stage(rs_flat, rsq_flat):
    mesh = plsc.VectorSubcoreMesh(core_axis_name="c", subcore_axis_name="s")

    @functools.partial(
        pl.kernel,
        mesh=mesh,
        out_type=[
            jax.ShapeDtypeStruct((B, L), jnp.float32),
            jax.ShapeDtypeStruct((B * T,), jnp.float32),
        ],
        scratch_types=[
            pltpu.VMEM((SH + 2 * L,), jnp.float32),
            pltpu.VMEM((SH,), jnp.float32),
            pltpu.VMEM((T,), jnp.float32),
            pltpu.VMEM((T,), jnp.float32),
            pltpu.VMEM((L,), jnp.float32),
            pltpu.VMEM((L,), jnp.int32),
            pltpu.VMEM((NSHARD * L,), jnp.float32),
            pltpu.VMEM((NSHARD * L,), jnp.int32),
            pltpu.VMEM_SHARED((16 * L,), jnp.float32),
            pltpu.VMEM_SHARED((16 * L,), jnp.int32),
            pltpu.VMEM_SHARED((2 * L,), jnp.int32),
        ],
        compiler_params=pltpu.CompilerParams(needs_layout_passes=False),
    )
    def sc_kernel(rs_hbm, rsq_hbm, stats_hbm, mask_hbm,
                  halo_v, xp_v, rs_full, rsq_full, stg_f, stg_i,
                  mv, mi, shv, shi, shp):
        c = lax.axis_index("c")
        s = lax.axis_index("s")
        b = c * 2 + s // NSHARD
        shard = s % NSHARD
        t0 = shard * SH
        base = b * T + t0
        lanes = lax.iota(jnp.int32, L)
        neg_inf = jnp.float32(-jnp.inf)
        big_i = jnp.int32(2**30)

        halo_v[pl.ds(0, L)] = jnp.full((L,), neg_inf, jnp.float32)
        halo_v[pl.ds(SH + L, L)] = jnp.full((L,), neg_inf, jnp.float32)
        pltpu.sync_copy(rs_hbm.at[pl.ds(base, SH)], halo_v.at[pl.ds(L, SH)])

        @pl.when(shard > 0)
        def _():
            pltpu.sync_copy(rs_hbm.at[pl.ds(base - L, L)],
                            halo_v.at[pl.ds(0, L)])

        @pl.when(shard < NSHARD - 1)
        def _():
            pltpu.sync_copy(rs_hbm.at[pl.ds(base + SH, L)],
                            halo_v.at[pl.ds(SH + L, L)])

        def peak_body(cb, carry):
            pos = cb * L + lanes
            ctr = plsc.load_gather(halo_v, [pos + L])
            lft = plsc.load_gather(halo_v, [pos + (L - 1)])
            rgt = plsc.load_gather(halo_v, [pos + (L + 1)])
            xp = jnp.where((ctr >= lft) & (ctr >= rgt), ctr, jnp.float32(0.0))
            plsc.store_scatter(xp_v, [pos], xp)
            return carry

        lax.fori_loop(0, NCHUNK, peak_body, 0)

        topv = jnp.full((L,), neg_inf, jnp.float32)
        topi = jnp.zeros((L,), jnp.int32)
        for k in range(NUM_PATTERN):
            def amax_body(cb, carry):
                m, mi_ = carry
                pos = cb * L + lanes
                v = plsc.load_gather(xp_v, [pos])
                gi = t0 + pos
                upd = (v > m) | ((v == m) & (gi < mi_))
                return jnp.where(upd, v, m), jnp.where(upd, gi, mi_)

            m, mi_ = lax.fori_loop(
                0, NCHUNK, amax_body,
                (jnp.full((L,), neg_inf, jnp.float32),
                 jnp.full((L,), big_i, jnp.int32)))
            maxv = jnp.max(m)
            gidx = jnp.min(jnp.where(m == maxv, mi_, big_i))
            topv = jnp.where(lanes == k, maxv, topv)
            topi = jnp.where(lanes == k, gidx, topi)
            plsc.store_scatter(xp_v, [jnp.zeros((L,), jnp.int32) + (gidx - t0)],
                               jnp.full((L,), neg_inf, jnp.float32),
                               mask=lanes == 0)

        stg_f[...] = topv
        stg_i[...] = topi
        pltpu.sync_copy(stg_f, shv.at[pl.ds(s * L, L)])
        pltpu.sync_copy(stg_i, shi.at[pl.ds(s * L, L)])
        plsc.subcore_barrier()

        @pl.when(shard == 0)
        def _():
            pltpu.sync_copy(shv.at[pl.ds((s // NSHARD) * NSHARD * L, NSHARD * L)], mv)
            pltpu.sync_copy(shi.at[pl.ds((s // NSHARD) * NSHARD * L, NSHARD * L)], mi)

            gtopv = jnp.full((L,), neg_inf, jnp.float32)
            gtopi = jnp.zeros((L,), jnp.int32)
            for k in range(NUM_PATTERN):
                def mrg_body(cb, carry):
                    m, mi_ = carry
                    pos = cb * L + lanes
                    v = plsc.load_gather(mv, [pos])
                    gi = plsc.load_gather(mi, [pos])
                    upd = (v > m) | ((v == m) & (gi < mi_))
                    return jnp.where(upd, v, m), jnp.where(upd, gi, mi_)

                m, mi_ = lax.fori_loop(
                    0, NSHARD, mrg_body,
                    (jnp.full((L,), neg_inf, jnp.float32),
                     jnp.full((L,), big_i, jnp.int32)))
                maxv = jnp.max(m)
                gidx = jnp.min(jnp.where(m == maxv, mi_, big_i))
                gtopv = jnp.where(lanes == k, maxv, gtopv)
                gtopi = jnp.where(lanes == k, gidx, gtopi)

                def clr_body(cb, carry):
                    pos = cb * L + lanes
                    v = plsc.load_gather(mv, [pos])
                    gi = plsc.load_gather(mi, [pos])
                    hit = (v == maxv) & (gi == gidx)
                    plsc.store_scatter(mv, [pos],
                                       jnp.full((L,), neg_inf, jnp.float32),
                                       mask=hit)
                    return carry

                lax.fori_loop(0, NSHARD, clr_body, 0)

            pltpu.sync_copy(rs_hbm.at[pl.ds(b * T, T)], rs_full)
            pltpu.sync_copy(rsq_hbm.at[pl.ds(b * T, T)], rsq_full)
            acc1 = jnp.zeros((L,), jnp.float32)
            acc2 = jnp.zeros((L,), jnp.float32)
            for k in range(NUM_PATTERN):
                pk = jnp.max(jnp.where(lanes == k, gtopi,
                                       jnp.int32(-2**31 + 1)))

                def win_body(jc, carry):
                    a1, a2 = carry
                    idxv = jnp.clip(pk - HALF + jc * L + lanes, 0, T - 1)
                    a1 = a1 + plsc.load_gather(rs_full, [idxv])
                    a2 = a2 + plsc.load_gather(rsq_full, [idxv])
                    return a1, a2

                acc1, acc2 = lax.fori_loop(0, PATTERN_LEN // L, win_body,
                                           (acc1, acc2))
            s1 = jnp.sum(acc1)
            s2 = jnp.sum(acc2)

            stg_f[...] = jnp.where(lanes == 0, s1,
                                   jnp.where(lanes == 1, s2,
                                             jnp.float32(0.0)))
            stg_i[...] = gtopi
            pltpu.sync_copy(stg_f, stats_hbm.at[b])
            pltpu.sync_copy(stg_i, shp.at[pl.ds((s // NSHARD) * L, L)])

        plsc.subcore_barrier()
        pltpu.sync_copy(shp.at[pl.ds((s // NSHARD) * L, L)], stg_i)
        pks = stg_i[...]
        los = []
        his = []
        for k in range(NUM_PATTERN):
            pk = jnp.max(jnp.where(lanes == k, pks, jnp.int32(-2**31 + 1)))
            los.append(jnp.maximum(pk - HALF, 0))
            his.append(jnp.minimum(pk + (PATTERN_LEN - 1 - HALF), T - 1))

        def mask_body(cb, carry):
            pos = cb * L + lanes
            t = t0 + pos
            inb = (t >= los[0]) & (t <= his[0])
            for k in range(1, NUM_PATTERN):
                inb = inb | ((t >= los[k]) & (t <= his[k]))
            plsc.store_scatter(
                xp_v, [pos],
                jnp.where(inb, jnp.float32(1.0), jnp.float32(0.0)))
            return carry

        lax.fori_loop(0, NCHUNK, mask_body, 0)
        pltpu.sync_copy(xp_v, mask_hbm.at[pl.ds(base, SH)])

    return sc_kernel(rs_flat, rsq_flat)


def _normalize(x, gamma2, beta2, stats, mask01):
    TTB = 2048

    def body(stats_ref, x_ref, m_ref, g_ref, bt_ref, y_ref):
        b = pl.program_id(0)
        s1 = stats_ref[b, 0]
        s2 = stats_ref[b, 1]
        n = jnp.float32(N_COMP)
        cmean = s1 / n
        var = (s2 - s1 * s1 / n) / jnp.float32(N_COMP - 1)
        inv = 1.0 / (jnp.sqrt(var) + jnp.float32(EPS))
        scale = g_ref[...] * inv
        bias = bt_ref[...] - cmean * scale
        c_in = jax.nn.sigmoid(jnp.float32(2.0))
        c_out = jax.nn.sigmoid(jnp.float32(-3.0))
        mcol = m_ref[...].reshape(TTB, 1)
        m = c_out + mcol * (c_in - c_out)
        xb = x_ref[0]
        y_ref[0] = (xb * scale + bias) * m

    return pl.pallas_call(
        body,
        grid=(B, T // TTB),
        in_specs=[
            pl.BlockSpec(memory_space=pltpu.SMEM),
            pl.BlockSpec((1, TTB, D), lambda b, i: (b, i, 0)),
            pl.BlockSpec((TTB,), lambda b, i: (b * (T // TTB) + i,)),
            pl.BlockSpec((1, D), lambda b, i: (0, 0)),
            pl.BlockSpec((1, D), lambda b, i: (0, 0)),
        ],
        out_specs=pl.BlockSpec((1, TTB, D), lambda b, i: (b, i, 0)),
        out_shape=jax.ShapeDtypeStruct((B, T, D), jnp.float32),
        compiler_params=pltpu.CompilerParams(
            dimension_semantics=("parallel", "parallel")),
    )(stats, x, mask01, gamma2, beta2)


def kernel(x, gamma, beta):
    rs, rsq = _row_stats(x)
    stats, mask01 = _sc_stage(rs, rsq)
    return _normalize(x, gamma.reshape(1, D), beta.reshape(1, D),
                      stats, mask01)


# speedup vs baseline: 1.6549x; 1.0108x over previous
import functools

import jax
import jax.numpy as jnp
from jax import lax
from jax.experimental import pallas as pl
from jax.experimental.pallas import tpu as pltpu
from jax.experimental.pallas import tpu_sc as plsc

B, T, D = 4, 8192, 1024
NUM_PATTERN = 8
PATTERN_LEN = T // 4 // NUM_PATTERN
HALF = PATTERN_LEN // 2
N_COMP = NUM_PATTERN * PATTERN_LEN * D
EPS = 1e-8

L = 16
NSHARD = 8
SH = T // NSHARD
NCHUNK = SH // L


def _row_stats(x):
    TTA = 2048

    def body(x_ref, rs_ref, rsq_ref):
        xb = x_ref[0]
        rs_ref[...] = jnp.sum(xb, axis=1)
        rsq_ref[...] = jnp.sum(xb * xb, axis=1)

    return pl.pallas_call(
        body,
        grid=(B, T // TTA),
        in_specs=[pl.BlockSpec((1, TTA, D), lambda b, i: (b, i, 0))],
        out_specs=[
            pl.BlockSpec((TTA,), lambda b, i: (b * (T // TTA) + i,)),
            pl.BlockSpec((TTA,), lambda b, i: (b * (T // TTA) + i,)),
        ],
        out_shape=[
            jax.ShapeDtypeStruct((B * T,), jnp.float32),
            jax.ShapeDtypeStruct((B * T,), jnp.float32),
        ],
        compiler_params=pltpu.CompilerParams(
            dimension_semantics=("parallel", "parallel")),
    )(x)


def _sc_stage(rs_flat, rsq_flat):
    mesh = plsc.VectorSubcoreMesh(core_axis_name="c", subcore_axis_name="s")

    @functools.partial(
        pl.kernel,
        mesh=mesh,
        out_type=[
            jax.ShapeDtypeStruct((B, L), jnp.float32),
            jax.ShapeDtypeStruct((B * T,), jnp.float32),
        ],
        scratch_types=[
            pltpu.VMEM((SH + 2 * L,), jnp.float32),
            pltpu.VMEM((SH,), jnp.float32),
            pltpu.VMEM((T,), jnp.float32),
            pltpu.VMEM((T,), jnp.float32),
            pltpu.VMEM((L,), jnp.float32),
            pltpu.VMEM((L,), jnp.int32),
            pltpu.VMEM((NSHARD * L,), jnp.float32),
            pltpu.VMEM((NSHARD * L,), jnp.int32),
            pltpu.VMEM_SHARED((16 * L,), jnp.float32),
            pltpu.VMEM_SHARED((16 * L,), jnp.int32),
            pltpu.VMEM_SHARED((2 * L,), jnp.int32),
        ],
        compiler_params=pltpu.CompilerParams(needs_layout_passes=False),
    )
    def sc_kernel(rs_hbm, rsq_hbm, stats_hbm, mask_hbm,
                  halo_v, xp_v, rs_full, rsq_full, stg_f, stg_i,
                  mv, mi, shv, shi, shp):
        c = lax.axis_index("c")
        s = lax.axis_index("s")
        b = c * 2 + s // NSHARD
        shard = s % NSHARD
        t0 = shard * SH
        base = b * T + t0
        lanes = lax.iota(jnp.int32, L)
        neg_inf = jnp.float32(-jnp.inf)
        big_i = jnp.int32(2**30)

        halo_v[pl.ds(0, L)] = jnp.full((L,), neg_inf, jnp.float32)
        halo_v[pl.ds(SH + L, L)] = jnp.full((L,), neg_inf, jnp.float32)
        pltpu.sync_copy(rs_hbm.at[pl.ds(base, SH)], halo_v.at[pl.ds(L, SH)])

        @pl.when(shard > 0)
        def _():
            pltpu.sync_copy(rs_hbm.at[pl.ds(base - L, L)],
                            halo_v.at[pl.ds(0, L)])

        @pl.when(shard < NSHARD - 1)
        def _():
            pltpu.sync_copy(rs_hbm.at[pl.ds(base + SH, L)],
                            halo_v.at[pl.ds(SH + L, L)])

        def peak_body(cb, carry):
            m, mi_ = carry
            pos = cb * L + lanes
            ctr = plsc.load_gather(halo_v, [pos + L])
            lft = plsc.load_gather(halo_v, [pos + (L - 1)])
            rgt = plsc.load_gather(halo_v, [pos + (L + 1)])
            xp = jnp.where((ctr >= lft) & (ctr >= rgt), ctr, jnp.float32(0.0))
            plsc.store_scatter(xp_v, [pos], xp)
            gi = t0 + pos
            upd = (xp > m) | ((xp == m) & (gi < mi_))
            return jnp.where(upd, xp, m), jnp.where(upd, gi, mi_)

        m, mi_ = lax.fori_loop(
            0, NCHUNK, peak_body,
            (jnp.full((L,), neg_inf, jnp.float32),
             jnp.full((L,), big_i, jnp.int32)))

        topv = jnp.full((L,), neg_inf, jnp.float32)
        topi = jnp.zeros((L,), jnp.int32)
        for k in range(NUM_PATTERN):
            maxv = jnp.max(m)
            gidx = jnp.min(jnp.where(m == maxv, mi_, big_i))
            topv = jnp.where(lanes == k, maxv, topv)
            topi = jnp.where(lanes == k, gidx, topi)
            plsc.store_scatter(xp_v, [jnp.zeros((L,), jnp.int32) + (gidx - t0)],
                               jnp.full((L,), neg_inf, jnp.float32),
                               mask=lanes == 0)
            if k == NUM_PATTERN - 1:
                break
            lstar = (gidx - t0) & (L - 1)
            mm = jnp.full((L,), neg_inf, jnp.float32)
            mii = jnp.full((L,), big_i, jnp.int32)
            for j in range(NCHUNK // L):
                pos = (j * L + lanes) * L + lstar
                v = plsc.load_gather(xp_v, [pos])
                gi = t0 + pos
                upd = (v > mm) | ((v == mm) & (gi < mii))
                mm = jnp.where(upd, v, mm)
                mii = jnp.where(upd, gi, mii)
            maxv_l = jnp.max(mm)
            gidx_l = jnp.min(jnp.where(mm == maxv_l, mii, big_i))
            m = jnp.where(lanes == lstar, maxv_l, m)
            mi_ = jnp.where(lanes == lstar, gidx_l, mi_)

        stg_f[...] = topv
        stg_i[...] = topi
        pltpu.sync_copy(stg_f, shv.at[pl.ds(s * L, L)])
        pltpu.sync_copy(stg_i, shi.at[pl.ds(s * L, L)])
        plsc.subcore_barrier()

        @pl.when(shard == 0)
        def _():
            pltpu.sync_copy(shv.at[pl.ds((s // NSHARD) * NSHARD * L, NSHARD * L)], mv)
            pltpu.sync_copy(shi.at[pl.ds((s // NSHARD) * NSHARD * L, NSHARD * L)], mi)

            gtopv = jnp.full((L,), neg_inf, jnp.float32)
            gtopi = jnp.zeros((L,), jnp.int32)
            for k in range(NUM_PATTERN):
                def mrg_body(cb, carry):
                    m, mi_ = carry
                    pos = cb * L + lanes
                    v = plsc.load_gather(mv, [pos])
                    gi = plsc.load_gather(mi, [pos])
                    upd = (v > m) | ((v == m) & (gi < mi_))
                    return jnp.where(upd, v, m), jnp.where(upd, gi, mi_)

                m, mi_ = lax.fori_loop(
                    0, NSHARD, mrg_body,
                    (jnp.full((L,), neg_inf, jnp.float32),
                     jnp.full((L,), big_i, jnp.int32)))
                maxv = jnp.max(m)
                gidx = jnp.min(jnp.where(m == maxv, mi_, big_i))
                gtopv = jnp.where(lanes == k, maxv, gtopv)
                gtopi = jnp.where(lanes == k, gidx, gtopi)

                def clr_body(cb, carry):
                    pos = cb * L + lanes
                    v = plsc.load_gather(mv, [pos])
                    gi = plsc.load_gather(mi, [pos])
                    hit = (v == maxv) & (gi == gidx)
                    plsc.store_scatter(mv, [pos],
                                       jnp.full((L,), neg_inf, jnp.float32),
                                       mask=hit)
                    return carry

                lax.fori_loop(0, NSHARD, clr_body, 0)

            pltpu.sync_copy(rs_hbm.at[pl.ds(b * T, T)], rs_full)
            pltpu.sync_copy(rsq_hbm.at[pl.ds(b * T, T)], rsq_full)
            acc1 = jnp.zeros((L,), jnp.float32)
            acc2 = jnp.zeros((L,), jnp.float32)
            for k in range(NUM_PATTERN):
                pk = jnp.max(jnp.where(lanes == k, gtopi,
                                       jnp.int32(-2**31 + 1)))

                def win_body(jc, carry):
                    a1, a2 = carry
                    idxv = jnp.clip(pk - HALF + jc * L + lanes, 0, T - 1)
                    a1 = a1 + plsc.load_gather(rs_full, [idxv])
                    a2 = a2 + plsc.load_gather(rsq_full, [idxv])
                    return a1, a2

                acc1, acc2 = lax.fori_loop(0, PATTERN_LEN // L, win_body,
                                           (acc1, acc2))
            s1 = jnp.sum(acc1)
            s2 = jnp.sum(acc2)

            stg_f[...] = jnp.where(lanes == 0, s1,
                                   jnp.where(lanes == 1, s2,
                                             jnp.float32(0.0)))
            stg_i[...] = gtopi
            pltpu.sync_copy(stg_f, stats_hbm.at[b])
            pltpu.sync_copy(stg_i, shp.at[pl.ds((s // NSHARD) * L, L)])

        plsc.subcore_barrier()
        pltpu.sync_copy(shp.at[pl.ds((s // NSHARD) * L, L)], stg_i)
        pks = stg_i[...]
        los = []
        his = []
        for k in range(NUM_PATTERN):
            pk = jnp.max(jnp.where(lanes == k, pks, jnp.int32(-2**31 + 1)))
            los.append(jnp.maximum(pk - HALF, 0))
            his.append(jnp.minimum(pk + (PATTERN_LEN - 1 - HALF), T - 1))

        def mask_body(cb, carry):
            pos = cb * L + lanes
            t = t0 + pos
            inb = (t >= los[0]) & (t <= his[0])
            for k in range(1, NUM_PATTERN):
                inb = inb | ((t >= los[k]) & (t <= his[k]))
            plsc.store_scatter(
                xp_v, [pos],
                jnp.where(inb, jnp.float32(1.0), jnp.float32(0.0)))
            return carry

        lax.fori_loop(0, NCHUNK, mask_body, 0)
        pltpu.sync_copy(xp_v, mask_hbm.at[pl.ds(base, SH)])

    return sc_kernel(rs_flat, rsq_flat)


def _normalize(x, gamma2, beta2, stats, mask01):
    TTB = 2048

    def body(stats_ref, x_ref, m_ref, g_ref, bt_ref, y_ref):
        b = pl.program_id(0)
        s1 = stats_ref[b, 0]
        s2 = stats_ref[b, 1]
        n = jnp.float32(N_COMP)
        cmean = s1 / n
        var = (s2 - s1 * s1 / n) / jnp.float32(N_COMP - 1)
        inv = 1.0 / (jnp.sqrt(var) + jnp.float32(EPS))
        scale = g_ref[...] * inv
        bias = bt_ref[...] - cmean * scale
        c_in = jax.nn.sigmoid(jnp.float32(2.0))
        c_out = jax.nn.sigmoid(jnp.float32(-3.0))
        mcol = m_ref[...].reshape(TTB, 1)
        m = c_out + mcol * (c_in - c_out)
        xb = x_ref[0]
        y_ref[0] = (xb * scale + bias) * m

    return pl.pallas_call(
        body,
        grid=(B, T // TTB),
        in_specs=[
            pl.BlockSpec(memory_space=pltpu.SMEM),
            pl.BlockSpec((1, TTB, D), lambda b, i: (b, i, 0)),
            pl.BlockSpec((TTB,), lambda b, i: (b * (T // TTB) + i,)),
            pl.BlockSpec((1, D), lambda b, i: (0, 0)),
            pl.BlockSpec((1, D), lambda b, i: (0, 0)),
        ],
        out_specs=pl.BlockSpec((1, TTB, D), lambda b, i: (b, i, 0)),
        out_shape=jax.ShapeDtypeStruct((B, T, D), jnp.float32),
        compiler_params=pltpu.CompilerParams(
            dimension_semantics=("parallel", "parallel")),
    )(stats, x, mask01, gamma2, beta2)


def kernel(x, gamma, beta):
    rs, rsq = _row_stats(x)
    stats, mask01 = _sc_stage(rs, rsq)
    return _normalize(x, gamma.reshape(1, D), beta.reshape(1, D),
                      stats, mask01)


# speedup vs baseline: 1.6902x; 1.0213x over previous
import functools

import jax
import jax.numpy as jnp
from jax import lax
from jax.experimental import pallas as pl
from jax.experimental.pallas import tpu as pltpu
from jax.experimental.pallas import tpu_sc as plsc

B, T, D = 4, 8192, 1024
NUM_PATTERN = 8
PATTERN_LEN = T // 4 // NUM_PATTERN
HALF = PATTERN_LEN // 2
N_COMP = NUM_PATTERN * PATTERN_LEN * D
EPS = 1e-8

L = 16
NSHARD = 8
SH = T // NSHARD
NCHUNK = SH // L


def _row_stats(x):
    TTA = 4096

    def body(x_ref, rs_ref, rsq_ref):
        xb = x_ref[0]
        rs_ref[...] = jnp.sum(xb, axis=1)
        rsq_ref[...] = jnp.sum(xb * xb, axis=1)

    return pl.pallas_call(
        body,
        grid=(B, T // TTA),
        in_specs=[pl.BlockSpec((1, TTA, D), lambda b, i: (b, i, 0))],
        out_specs=[
            pl.BlockSpec((TTA,), lambda b, i: (b * (T // TTA) + i,)),
            pl.BlockSpec((TTA,), lambda b, i: (b * (T // TTA) + i,)),
        ],
        out_shape=[
            jax.ShapeDtypeStruct((B * T,), jnp.float32),
            jax.ShapeDtypeStruct((B * T,), jnp.float32),
        ],
        compiler_params=pltpu.CompilerParams(
            dimension_semantics=("parallel", "parallel"),
            vmem_limit_bytes=100 * 1024 * 1024),
    )(x)


def _sc_stage(rs_flat, rsq_flat):
    mesh = plsc.VectorSubcoreMesh(core_axis_name="c", subcore_axis_name="s")

    @functools.partial(
        pl.kernel,
        mesh=mesh,
        out_type=[
            jax.ShapeDtypeStruct((B, L), jnp.float32),
            jax.ShapeDtypeStruct((B * T,), jnp.float32),
        ],
        scratch_types=[
            pltpu.VMEM((SH + 2 * L,), jnp.float32),
            pltpu.VMEM((SH,), jnp.float32),
            pltpu.VMEM((T,), jnp.float32),
            pltpu.VMEM((T,), jnp.float32),
            pltpu.VMEM((L,), jnp.float32),
            pltpu.VMEM((L,), jnp.int32),
            pltpu.VMEM((NSHARD * L,), jnp.float32),
            pltpu.VMEM((NSHARD * L,), jnp.int32),
            pltpu.VMEM_SHARED((16 * L,), jnp.float32),
            pltpu.VMEM_SHARED((16 * L,), jnp.int32),
            pltpu.VMEM_SHARED((2 * L,), jnp.int32),
        ],
        compiler_params=pltpu.CompilerParams(needs_layout_passes=False),
    )
    def sc_kernel(rs_hbm, rsq_hbm, stats_hbm, mask_hbm,
                  halo_v, xp_v, rs_full, rsq_full, stg_f, stg_i,
                  mv, mi, shv, shi, shp):
        c = lax.axis_index("c")
        s = lax.axis_index("s")
        b = c * 2 + s // NSHARD
        shard = s % NSHARD
        t0 = shard * SH
        base = b * T + t0
        lanes = lax.iota(jnp.int32, L)
        neg_inf = jnp.float32(-jnp.inf)
        big_i = jnp.int32(2**30)

        halo_v[pl.ds(0, L)] = jnp.full((L,), neg_inf, jnp.float32)
        halo_v[pl.ds(SH + L, L)] = jnp.full((L,), neg_inf, jnp.float32)
        pltpu.sync_copy(rs_hbm.at[pl.ds(base, SH)], halo_v.at[pl.ds(L, SH)])

        @pl.when(shard > 0)
        def _():
            pltpu.sync_copy(rs_hbm.at[pl.ds(base - L, L)],
                            halo_v.at[pl.ds(0, L)])

        @pl.when(shard < NSHARD - 1)
        def _():
            pltpu.sync_copy(rs_hbm.at[pl.ds(base + SH, L)],
                            halo_v.at[pl.ds(SH + L, L)])

        def peak_body(cb, carry):
            m, mi_ = carry
            pos = cb * L + lanes
            ctr = plsc.load_gather(halo_v, [pos + L])
            lft = plsc.load_gather(halo_v, [pos + (L - 1)])
            rgt = plsc.load_gather(halo_v, [pos + (L + 1)])
            xp = jnp.where((ctr >= lft) & (ctr >= rgt), ctr, jnp.float32(0.0))
            plsc.store_scatter(xp_v, [pos], xp)
            gi = t0 + pos
            upd = (xp > m) | ((xp == m) & (gi < mi_))
            return jnp.where(upd, xp, m), jnp.where(upd, gi, mi_)

        m, mi_ = lax.fori_loop(
            0, NCHUNK, peak_body,
            (jnp.full((L,), neg_inf, jnp.float32),
             jnp.full((L,), big_i, jnp.int32)))

        topv = jnp.full((L,), neg_inf, jnp.float32)
        topi = jnp.zeros((L,), jnp.int32)
        for k in range(NUM_PATTERN):
            maxv = jnp.max(m)
            gidx = jnp.min(jnp.where(m == maxv, mi_, big_i))
            topv = jnp.where(lanes == k, maxv, topv)
            topi = jnp.where(lanes == k, gidx, topi)
            plsc.store_scatter(xp_v, [jnp.zeros((L,), jnp.int32) + (gidx - t0)],
                               jnp.full((L,), neg_inf, jnp.float32),
                               mask=lanes == 0)
            if k == NUM_PATTERN - 1:
                break
            lstar = (gidx - t0) & (L - 1)
            mm = jnp.full((L,), neg_inf, jnp.float32)
            mii = jnp.full((L,), big_i, jnp.int32)
            for j in range(NCHUNK // L):
                pos = (j * L + lanes) * L + lstar
                v = plsc.load_gather(xp_v, [pos])
                gi = t0 + pos
                upd = (v > mm) | ((v == mm) & (gi < mii))
                mm = jnp.where(upd, v, mm)
                mii = jnp.where(upd, gi, mii)
            maxv_l = jnp.max(mm)
            gidx_l = jnp.min(jnp.where(mm == maxv_l, mii, big_i))
            m = jnp.where(lanes == lstar, maxv_l, m)
            mi_ = jnp.where(lanes == lstar, gidx_l, mi_)

        stg_f[...] = topv
        stg_i[...] = topi
        pltpu.sync_copy(stg_f, shv.at[pl.ds(s * L, L)])
        pltpu.sync_copy(stg_i, shi.at[pl.ds(s * L, L)])
        plsc.subcore_barrier()

        @pl.when(shard == 0)
        def _():
            pltpu.sync_copy(shv.at[pl.ds((s // NSHARD) * NSHARD * L, NSHARD * L)], mv)
            pltpu.sync_copy(shi.at[pl.ds((s // NSHARD) * NSHARD * L, NSHARD * L)], mi)

            gtopv = jnp.full((L,), neg_inf, jnp.float32)
            gtopi = jnp.zeros((L,), jnp.int32)
            for k in range(NUM_PATTERN):
                def mrg_body(cb, carry):
                    m, mi_ = carry
                    pos = cb * L + lanes
                    v = plsc.load_gather(mv, [pos])
                    gi = plsc.load_gather(mi, [pos])
                    upd = (v > m) | ((v == m) & (gi < mi_))
                    return jnp.where(upd, v, m), jnp.where(upd, gi, mi_)

                m, mi_ = lax.fori_loop(
                    0, NSHARD, mrg_body,
                    (jnp.full((L,), neg_inf, jnp.float32),
                     jnp.full((L,), big_i, jnp.int32)))
                maxv = jnp.max(m)
                gidx = jnp.min(jnp.where(m == maxv, mi_, big_i))
                gtopv = jnp.where(lanes == k, maxv, gtopv)
                gtopi = jnp.where(lanes == k, gidx, gtopi)

                def clr_body(cb, carry):
                    pos = cb * L + lanes
                    v = plsc.load_gather(mv, [pos])
                    gi = plsc.load_gather(mi, [pos])
                    hit = (v == maxv) & (gi == gidx)
                    plsc.store_scatter(mv, [pos],
                                       jnp.full((L,), neg_inf, jnp.float32),
                                       mask=hit)
                    return carry

                lax.fori_loop(0, NSHARD, clr_body, 0)

            pltpu.sync_copy(rs_hbm.at[pl.ds(b * T, T)], rs_full)
            pltpu.sync_copy(rsq_hbm.at[pl.ds(b * T, T)], rsq_full)
            acc1 = jnp.zeros((L,), jnp.float32)
            acc2 = jnp.zeros((L,), jnp.float32)
            for k in range(NUM_PATTERN):
                pk = jnp.max(jnp.where(lanes == k, gtopi,
                                       jnp.int32(-2**31 + 1)))

                def win_body(jc, carry):
                    a1, a2 = carry
                    idxv = jnp.clip(pk - HALF + jc * L + lanes, 0, T - 1)
                    a1 = a1 + plsc.load_gather(rs_full, [idxv])
                    a2 = a2 + plsc.load_gather(rsq_full, [idxv])
                    return a1, a2

                acc1, acc2 = lax.fori_loop(0, PATTERN_LEN // L, win_body,
                                           (acc1, acc2))
            s1 = jnp.sum(acc1)
            s2 = jnp.sum(acc2)

            stg_f[...] = jnp.where(lanes == 0, s1,
                                   jnp.where(lanes == 1, s2,
                                             jnp.float32(0.0)))
            stg_i[...] = gtopi
            pltpu.sync_copy(stg_f, stats_hbm.at[b])
            pltpu.sync_copy(stg_i, shp.at[pl.ds((s // NSHARD) * L, L)])

        plsc.subcore_barrier()
        pltpu.sync_copy(shp.at[pl.ds((s // NSHARD) * L, L)], stg_i)
        pks = stg_i[...]
        los = []
        his = []
        for k in range(NUM_PATTERN):
            pk = jnp.max(jnp.where(lanes == k, pks, jnp.int32(-2**31 + 1)))
            los.append(jnp.maximum(pk - HALF, 0))
            his.append(jnp.minimum(pk + (PATTERN_LEN - 1 - HALF), T - 1))

        def mask_body(cb, carry):
            pos = cb * L + lanes
            t = t0 + pos
            inb = (t >= los[0]) & (t <= his[0])
            for k in range(1, NUM_PATTERN):
                inb = inb | ((t >= los[k]) & (t <= his[k]))
            plsc.store_scatter(
                xp_v, [pos],
                jnp.where(inb, jnp.float32(1.0), jnp.float32(0.0)))
            return carry

        lax.fori_loop(0, NCHUNK, mask_body, 0)
        pltpu.sync_copy(xp_v, mask_hbm.at[pl.ds(base, SH)])

    return sc_kernel(rs_flat, rsq_flat)


def _normalize(x, gamma2, beta2, stats, mask01):
    TTB = 2048

    def body(stats_ref, x_ref, m_ref, g_ref, bt_ref, y_ref):
        b = pl.program_id(0)
        s1 = stats_ref[b, 0]
        s2 = stats_ref[b, 1]
        n = jnp.float32(N_COMP)
        cmean = s1 / n
        var = (s2 - s1 * s1 / n) / jnp.float32(N_COMP - 1)
        inv = 1.0 / (jnp.sqrt(var) + jnp.float32(EPS))
        scale = g_ref[...] * inv
        bias = bt_ref[...] - cmean * scale
        c_in = jax.nn.sigmoid(jnp.float32(2.0))
        c_out = jax.nn.sigmoid(jnp.float32(-3.0))
        mcol = m_ref[...].reshape(TTB, 1)
        m = c_out + mcol * (c_in - c_out)
        xb = x_ref[0]
        y_ref[0] = (xb * scale + bias) * m

    return pl.pallas_call(
        body,
        grid=(B, T // TTB),
        in_specs=[
            pl.BlockSpec(memory_space=pltpu.SMEM),
            pl.BlockSpec((1, TTB, D), lambda b, i: (b, i, 0)),
            pl.BlockSpec((TTB,), lambda b, i: (b * (T // TTB) + i,)),
            pl.BlockSpec((1, D), lambda b, i: (0, 0)),
            pl.BlockSpec((1, D), lambda b, i: (0, 0)),
        ],
        out_specs=pl.BlockSpec((1, TTB, D), lambda b, i: (b, i, 0)),
        out_shape=jax.ShapeDtypeStruct((B, T, D), jnp.float32),
        compiler_params=pltpu.CompilerParams(
            dimension_semantics=("parallel", "parallel")),
    )(stats, x, mask01, gamma2, beta2)


def kernel(x, gamma, beta):
    rs, rsq = _row_stats(x)
    stats, mask01 = _sc_stage(rs, rsq)
    return _normalize(x, gamma.reshape(1, D), beta.reshape(1, D),
                      stats, mask01)


# speedup vs baseline: 1.7037x; 1.0080x over previous
import functools

import jax
import jax.numpy as jnp
from jax import lax
from jax.experimental import pallas as pl
from jax.experimental.pallas import tpu as pltpu
from jax.experimental.pallas import tpu_sc as plsc

B, T, D = 4, 8192, 1024
NUM_PATTERN = 8
PATTERN_LEN = T // 4 // NUM_PATTERN
HALF = PATTERN_LEN // 2
N_COMP = NUM_PATTERN * PATTERN_LEN * D
EPS = 1e-8

L = 16
NSHARD = 8
SH = T // NSHARD
NCHUNK = SH // L
WIN = 272


def _row_stats(x):
    TTA = 4096

    def body(x_ref, rs_ref, rsq_ref):
        xb = x_ref[0]
        rs_ref[...] = jnp.sum(xb, axis=1)
        rsq_ref[...] = jnp.sum(xb * xb, axis=1)

    return pl.pallas_call(
        body,
        grid=(B, T // TTA),
        in_specs=[pl.BlockSpec((1, TTA, D), lambda b, i: (b, i, 0))],
        out_specs=[
            pl.BlockSpec((TTA,), lambda b, i: (b * (T // TTA) + i,)),
            pl.BlockSpec((TTA,), lambda b, i: (b * (T // TTA) + i,)),
        ],
        out_shape=[
            jax.ShapeDtypeStruct((B * T,), jnp.float32),
            jax.ShapeDtypeStruct((B * T,), jnp.float32),
        ],
        compiler_params=pltpu.CompilerParams(
            dimension_semantics=("parallel", "parallel"),
            vmem_limit_bytes=100 * 1024 * 1024),
    )(x)


def _sc_stage(rs_flat, rsq_flat):
    mesh = plsc.VectorSubcoreMesh(core_axis_name="c", subcore_axis_name="s")

    @functools.partial(
        pl.kernel,
        mesh=mesh,
        out_type=[
            jax.ShapeDtypeStruct((B, L), jnp.float32),
            jax.ShapeDtypeStruct((B * T,), jnp.float32),
        ],
        scratch_types=[
            pltpu.VMEM((SH + 2 * L,), jnp.float32),
            pltpu.VMEM((SH,), jnp.float32),
            pltpu.VMEM((WIN,), jnp.float32),
            pltpu.VMEM((WIN,), jnp.float32),
            pltpu.VMEM((L,), jnp.float32),
            pltpu.VMEM((L,), jnp.int32),
            pltpu.VMEM((NSHARD * L,), jnp.float32),
            pltpu.VMEM((NSHARD * L,), jnp.int32),
            pltpu.VMEM((NSHARD * L,), jnp.float32),
            pltpu.VMEM_SHARED((16 * L,), jnp.float32),
            pltpu.VMEM_SHARED((16 * L,), jnp.int32),
            pltpu.VMEM_SHARED((2 * L,), jnp.int32),
            pltpu.VMEM_SHARED((16 * L,), jnp.float32),
            pltpu.VMEM_SHARED((16 * L,), jnp.float32),
        ],
        compiler_params=pltpu.CompilerParams(needs_layout_passes=False),
    )
    def sc_kernel(rs_hbm, rsq_hbm, stats_hbm, mask_hbm,
                  halo_v, xp_v, wbuf1, wbuf2, stg_f, stg_i,
                  mv, mi, mv2, shv, shi, shp, shw1, shw2):
        c = lax.axis_index("c")
        s = lax.axis_index("s")
        b = c * 2 + s // NSHARD
        shard = s % NSHARD
        t0 = shard * SH
        base = b * T + t0
        lanes = lax.iota(jnp.int32, L)
        neg_inf = jnp.float32(-jnp.inf)
        big_i = jnp.int32(2**30)

        halo_v[pl.ds(0, L)] = jnp.full((L,), neg_inf, jnp.float32)
        halo_v[pl.ds(SH + L, L)] = jnp.full((L,), neg_inf, jnp.float32)
        pltpu.sync_copy(rs_hbm.at[pl.ds(base, SH)], halo_v.at[pl.ds(L, SH)])

        @pl.when(shard > 0)
        def _():
            pltpu.sync_copy(rs_hbm.at[pl.ds(base - L, L)],
                            halo_v.at[pl.ds(0, L)])

        @pl.when(shard < NSHARD - 1)
        def _():
            pltpu.sync_copy(rs_hbm.at[pl.ds(base + SH, L)],
                            halo_v.at[pl.ds(SH + L, L)])

        def peak_body(cb, carry):
            m, mi_ = carry
            pos = cb * L + lanes
            ctr = plsc.load_gather(halo_v, [pos + L])
            lft = plsc.load_gather(halo_v, [pos + (L - 1)])
            rgt = plsc.load_gather(halo_v, [pos + (L + 1)])
            xp = jnp.where((ctr >= lft) & (ctr >= rgt), ctr, jnp.float32(0.0))
            plsc.store_scatter(xp_v, [pos], xp)
            gi = t0 + pos
            upd = (xp > m) | ((xp == m) & (gi < mi_))
            return jnp.where(upd, xp, m), jnp.where(upd, gi, mi_)

        m, mi_ = lax.fori_loop(
            0, NCHUNK, peak_body,
            (jnp.full((L,), neg_inf, jnp.float32),
             jnp.full((L,), big_i, jnp.int32)))

        topv = jnp.full((L,), neg_inf, jnp.float32)
        topi = jnp.zeros((L,), jnp.int32)
        for k in range(NUM_PATTERN):
            maxv = jnp.max(m)
            gidx = jnp.min(jnp.where(m == maxv, mi_, big_i))
            topv = jnp.where(lanes == k, maxv, topv)
            topi = jnp.where(lanes == k, gidx, topi)
            plsc.store_scatter(xp_v, [jnp.zeros((L,), jnp.int32) + (gidx - t0)],
                               jnp.full((L,), neg_inf, jnp.float32),
                               mask=lanes == 0)
            if k == NUM_PATTERN - 1:
                break
            lstar = (gidx - t0) & (L - 1)
            mm = jnp.full((L,), neg_inf, jnp.float32)
            mii = jnp.full((L,), big_i, jnp.int32)
            for j in range(NCHUNK // L):
                pos = (j * L + lanes) * L + lstar
                v = plsc.load_gather(xp_v, [pos])
                gi = t0 + pos
                upd = (v > mm) | ((v == mm) & (gi < mii))
                mm = jnp.where(upd, v, mm)
                mii = jnp.where(upd, gi, mii)
            maxv_l = jnp.max(mm)
            gidx_l = jnp.min(jnp.where(mm == maxv_l, mii, big_i))
            m = jnp.where(lanes == lstar, maxv_l, m)
            mi_ = jnp.where(lanes == lstar, gidx_l, mi_)

        stg_f[...] = topv
        stg_i[...] = topi
        pltpu.sync_copy(stg_f, shv.at[pl.ds(s * L, L)])
        pltpu.sync_copy(stg_i, shi.at[pl.ds(s * L, L)])
        plsc.subcore_barrier()

        @pl.when(shard == 0)
        def _():
            pltpu.sync_copy(shv.at[pl.ds((s // NSHARD) * NSHARD * L, NSHARD * L)], mv)
            pltpu.sync_copy(shi.at[pl.ds((s // NSHARD) * NSHARD * L, NSHARD * L)], mi)

            gtopv = jnp.full((L,), neg_inf, jnp.float32)
            gtopi = jnp.zeros((L,), jnp.int32)
            for k in range(NUM_PATTERN):
                def mrg_body(cb, carry):
                    m, mi_ = carry
                    pos = cb * L + lanes
                    v = plsc.load_gather(mv, [pos])
                    gi = plsc.load_gather(mi, [pos])
                    upd = (v > m) | ((v == m) & (gi < mi_))
                    return jnp.where(upd, v, m), jnp.where(upd, gi, mi_)

                m, mi_ = lax.fori_loop(
                    0, NSHARD, mrg_body,
                    (jnp.full((L,), neg_inf, jnp.float32),
                     jnp.full((L,), big_i, jnp.int32)))
                maxv = jnp.max(m)
                gidx = jnp.min(jnp.where(m == maxv, mi_, big_i))
                gtopv = jnp.where(lanes == k, maxv, gtopv)
                gtopi = jnp.where(lanes == k, gidx, gtopi)

                def clr_body(cb, carry):
                    pos = cb * L + lanes
                    v = plsc.load_gather(mv, [pos])
                    gi = plsc.load_gather(mi, [pos])
                    hit = (v == maxv) & (gi == gidx)
                    plsc.store_scatter(mv, [pos],
                                       jnp.full((L,), neg_inf, jnp.float32),
                                       mask=hit)
                    return carry

                lax.fori_loop(0, NSHARD, clr_body, 0)

            stg_i[...] = gtopi
            pltpu.sync_copy(stg_i, shp.at[pl.ds((s // NSHARD) * L, L)])

        plsc.subcore_barrier()
        pltpu.sync_copy(shp.at[pl.ds((s // NSHARD) * L, L)], stg_i)
        pks = stg_i[...]
        los = []
        his = []
        for k in range(NUM_PATTERN):
            pk = jnp.max(jnp.where(lanes == k, pks, jnp.int32(-2**31 + 1)))
            los.append(jnp.maximum(pk - HALF, 0))
            his.append(jnp.minimum(pk + (PATTERN_LEN - 1 - HALF), T - 1))

        pk_mine = jnp.max(jnp.where(lanes == shard, pks,
                                    jnp.int32(-2**31 + 1)))
        start = jnp.minimum(
            jnp.bitwise_and(jnp.maximum(pk_mine - HALF, 0), jnp.int32(-8)),
            jnp.int32(T - WIN))
        woff = pl.multiple_of(b * T + start, 8)
        pltpu.sync_copy(rs_hbm.at[pl.ds(woff, WIN)], wbuf1)
        pltpu.sync_copy(rsq_hbm.at[pl.ds(woff, WIN)], wbuf2)

        def win_body(jc, carry):
            a1, a2 = carry
            idxv = jnp.clip(pk_mine - HALF + jc * L + lanes, 0, T - 1) - start
            a1 = a1 + plsc.load_gather(wbuf1, [idxv])
            a2 = a2 + plsc.load_gather(wbuf2, [idxv])
            return a1, a2

        acc1, acc2 = lax.fori_loop(
            0, PATTERN_LEN // L, win_body,
            (jnp.zeros((L,), jnp.float32), jnp.zeros((L,), jnp.float32)))
        stg_f[...] = acc1
        pltpu.sync_copy(stg_f, shw1.at[pl.ds(s * L, L)])
        stg_f[...] = acc2
        pltpu.sync_copy(stg_f, shw2.at[pl.ds(s * L, L)])
        plsc.subcore_barrier()

        @pl.when(shard == 0)
        def _():
            base_row = (s // NSHARD) * NSHARD * L
            pltpu.sync_copy(shw1.at[pl.ds(base_row, NSHARD * L)], mv)
            pltpu.sync_copy(shw2.at[pl.ds(base_row, NSHARD * L)], mv2)

            def red_body(cb, carry):
                a1, a2 = carry
                pos = cb * L + lanes
                a1 = a1 + plsc.load_gather(mv, [pos])
                a2 = a2 + plsc.load_gather(mv2, [pos])
                return a1, a2

            a1, a2 = lax.fori_loop(
                0, NSHARD, red_body,
                (jnp.zeros((L,), jnp.float32), jnp.zeros((L,), jnp.float32)))
            s1 = jnp.sum(a1)
            s2 = jnp.sum(a2)
            stg_f[...] = jnp.where(lanes == 0, s1,
                                   jnp.where(lanes == 1, s2,
                                             jnp.float32(0.0)))
            pltpu.sync_copy(stg_f, stats_hbm.at[b])

        def mask_body(cb, carry):
            pos = cb * L + lanes
            t = t0 + pos
            inb = (t >= los[0]) & (t <= his[0])
            for k in range(1, NUM_PATTERN):
                inb = inb | ((t >= los[k]) & (t <= his[k]))
            plsc.store_scatter(
                xp_v, [pos],
                jnp.where(inb, jnp.float32(1.0), jnp.float32(0.0)))
            return carry

        lax.fori_loop(0, NCHUNK, mask_body, 0)
        pltpu.sync_copy(xp_v, mask_hbm.at[pl.ds(base, SH)])

    return sc_kernel(rs_flat, rsq_flat)


def _normalize(x, gamma2, beta2, stats, mask01):
    TTB = 2048

    def body(stats_ref, x_ref, m_ref, g_ref, bt_ref, y_ref):
        b = pl.program_id(0)
        s1 = stats_ref[b, 0]
        s2 = stats_ref[b, 1]
        n = jnp.float32(N_COMP)
        cmean = s1 / n
        var = (s2 - s1 * s1 / n) / jnp.float32(N_COMP - 1)
        inv = 1.0 / (jnp.sqrt(var) + jnp.float32(EPS))
        scale = g_ref[...] * inv
        bias = bt_ref[...] - cmean * scale
        c_in = jax.nn.sigmoid(jnp.float32(2.0))
        c_out = jax.nn.sigmoid(jnp.float32(-3.0))
        mcol = m_ref[...].reshape(TTB, 1)
        m = c_out + mcol * (c_in - c_out)
        xb = x_ref[0]
        y_ref[0] = (xb * scale + bias) * m

    return pl.pallas_call(
        body,
        grid=(B, T // TTB),
        in_specs=[
            pl.BlockSpec(memory_space=pltpu.SMEM),
            pl.BlockSpec((1, TTB, D), lambda b, i: (b, i, 0)),
            pl.BlockSpec((TTB,), lambda b, i: (b * (T // TTB) + i,)),
            pl.BlockSpec((1, D), lambda b, i: (0, 0)),
            pl.BlockSpec((1, D), lambda b, i: (0, 0)),
        ],
        out_specs=pl.BlockSpec((1, TTB, D), lambda b, i: (b, i, 0)),
        out_shape=jax.ShapeDtypeStruct((B, T, D), jnp.float32),
        compiler_params=pltpu.CompilerParams(
            dimension_semantics=("parallel", "parallel")),
    )(stats, x, mask01, gamma2, beta2)


def kernel(x, gamma, beta):
    rs, rsq = _row_stats(x)
    stats, mask01 = _sc_stage(rs, rsq)
    return _normalize(x, gamma.reshape(1, D), beta.reshape(1, D),
                      stats, mask01)
